# Initial kernel scaffold; baseline (speedup 1.0000x reference)
import jax
import jax.numpy as jnp
from jax.experimental import pallas as pl


def kernel(sku, category, event_type, sku_table, category_table, event_type_table):
    raise NotImplementedError("write your pallas kernel here")


# speedup vs baseline: 1.9529x; 1.9529x over previous
import functools

import jax
import jax.numpy as jnp
from jax import lax
from jax.experimental import pallas as pl
from jax.experimental.pallas import tpu as pltpu
from jax.experimental.pallas import tpu_sc as plsc

SKU_D = 64
CAT_D = 32
EVT_D = 16
OUT_D = SKU_D + CAT_D + EVT_D

NW = 32
CH = 128
CPB = 8
BLK = CH * CPB


@functools.lru_cache(maxsize=None)
def _make_kernel(R: int):
    per_w = R // NW
    nblk = per_w // BLK
    assert per_w % BLK == 0
    mesh = plsc.VectorSubcoreMesh(core_axis_name="c", subcore_axis_name="s")

    @functools.partial(
        pl.kernel,
        out_type=jax.ShapeDtypeStruct((R, OUT_D), jnp.float32),
        mesh=mesh,
        compiler_params=pltpu.CompilerParams(use_tc_tiling_on_sc=False),
        scratch_types=[
            pltpu.VMEM((CPB, CH), jnp.int32),
            pltpu.VMEM((CPB, CH), jnp.int32),
            pltpu.VMEM((CPB, CH), jnp.int32),
            pltpu.VMEM((BLK, SKU_D), jnp.float32),
            pltpu.VMEM((BLK, CAT_D), jnp.float32),
            pltpu.VMEM((BLK, EVT_D), jnp.float32),
            pltpu.SemaphoreType.DMA,
        ],
    )
    def k(sku_idx, cat_idx, evt_idx, sku_tab, cat_tab, evt_tab, out,
          sidx_v, cidx_v, eidx_v, sbuf, cbuf, ebuf, sem):
        wid = lax.axis_index("s") * 2 + lax.axis_index("c")
        w_base = wid * per_w

        @pl.loop(0, nblk)
        def _(b):
            base = w_base + b * BLK
            crow = base // CH
            pltpu.sync_copy(sku_idx.at[pl.ds(crow, CPB)], sidx_v)
            pltpu.sync_copy(cat_idx.at[pl.ds(crow, CPB)], cidx_v)
            pltpu.sync_copy(evt_idx.at[pl.ds(crow, CPB)], eidx_v)
            copies = []
            for j in range(CPB):
                copies.append(pltpu.async_copy(
                    sku_tab.at[sidx_v.at[j]], sbuf.at[pl.ds(j * CH, CH)], sem))
                copies.append(pltpu.async_copy(
                    cat_tab.at[cidx_v.at[j]], cbuf.at[pl.ds(j * CH, CH)], sem))
                copies.append(pltpu.async_copy(
                    evt_tab.at[eidx_v.at[j]], ebuf.at[pl.ds(j * CH, CH)], sem))
            for c in copies:
                c.wait()
            pltpu.sync_copy(sbuf, out.at[pl.ds(base, BLK), pl.ds(0, SKU_D)])
            pltpu.sync_copy(cbuf, out.at[pl.ds(base, BLK), pl.ds(SKU_D, CAT_D)])
            pltpu.sync_copy(
                ebuf, out.at[pl.ds(base, BLK), pl.ds(SKU_D + CAT_D, EVT_D)])

    return k


def kernel(sku, category, event_type, sku_table, category_table, event_type_table):
    B, H = sku.shape
    R = B * H
    sku_i = sku.reshape(R // CH, CH).astype(jnp.int32)
    cat_i = category.reshape(R // CH, CH).astype(jnp.int32)
    evt_i = event_type.reshape(R // CH, CH).astype(jnp.int32)
    out = _make_kernel(R)(
        sku_i, cat_i, evt_i,
        sku_table.astype(jnp.float32),
        category_table.astype(jnp.float32),
        event_type_table.astype(jnp.float32),
    )
    return out.reshape(B, H, OUT_D)


# speedup vs baseline: 3.7050x; 1.8972x over previous
import functools

import jax
import jax.numpy as jnp
from jax import lax
from jax.experimental import pallas as pl
from jax.experimental.pallas import tpu as pltpu
from jax.experimental.pallas import tpu_sc as plsc

SKU_D = 64
CAT_D = 32
EVT_D = 16
OUT_D = SKU_D + CAT_D + EVT_D
CAT_V = 1001
EVT_V = 11

NW = 32
CH = 128
BLK = 256
L = 16


@functools.lru_cache(maxsize=None)
def _make_kernel(R: int):
    per_w = R // NW
    nblk = per_w // BLK
    assert per_w % BLK == 0 and nblk % 2 == 0
    mesh = plsc.VectorSubcoreMesh(core_axis_name="c", subcore_axis_name="s")

    @functools.partial(
        pl.kernel,
        out_type=jax.ShapeDtypeStruct((R, OUT_D), jnp.float32),
        mesh=mesh,
        compiler_params=pltpu.CompilerParams(use_tc_tiling_on_sc=False, needs_layout_passes=False),
        scratch_types=[
            pltpu.VMEM((BLK,), jnp.int32),
            pltpu.VMEM((BLK,), jnp.int32),
            pltpu.VMEM((BLK,), jnp.int32),
            pltpu.VMEM((BLK,), jnp.int32),
            pltpu.VMEM((BLK,), jnp.int32),
            pltpu.VMEM((BLK,), jnp.int32),
            pltpu.VMEM((BLK, SKU_D), jnp.float32),
            pltpu.VMEM((BLK, SKU_D), jnp.float32),
            pltpu.VMEM((BLK, CAT_D + EVT_D), jnp.float32),
            pltpu.VMEM((BLK, CAT_D + EVT_D), jnp.float32),
            pltpu.VMEM((CAT_V * CAT_D,), jnp.float32),
            pltpu.VMEM((EVT_V * EVT_D,), jnp.float32),
            pltpu.SemaphoreType.DMA,
            pltpu.SemaphoreType.DMA,
            pltpu.SemaphoreType.DMA,
            pltpu.SemaphoreType.DMA,
        ],
    )
    def k(sidx_h, cidx_h, eidx_h, sku_tab, cat_tab, evt_tab, out,
          sidx0, sidx1, cidx0, cidx1, eidx0, eidx1, sbuf0, sbuf1,
          obuf0, obuf1, cat_v, evt_v, gsem0, gsem1, osem0, osem1):
        sidx = (sidx0, sidx1)
        cidx = (cidx0, cidx1)
        eidx = (eidx0, eidx1)
        sbuf = (sbuf0, sbuf1)
        obuf = (obuf0, obuf1)
        gsem = (gsem0, gsem1)
        osem = (osem0, osem1)

        wid = lax.axis_index("s") * 2 + lax.axis_index("c")
        w_base = wid * per_w

        pltpu.sync_copy(cat_tab, cat_v)
        pltpu.sync_copy(evt_tab, evt_v)

        rows0 = lax.iota(jnp.int32, L)

        @pl.loop(0, nblk, step=2)
        def _(b0):
            for p in range(2):
                b = b0 + p
                base = w_base + b * BLK

                @pl.when(b0 >= 2)
                def _():
                    pltpu.make_async_copy(
                        sbuf[p], out.at[pl.ds(0, BLK), pl.ds(0, SKU_D)],
                        osem[p]).wait()
                    pltpu.make_async_copy(
                        obuf[p], out.at[pl.ds(0, BLK), pl.ds(SKU_D, CAT_D + EVT_D)],
                        osem[p]).wait()

                pltpu.sync_copy(sidx_h.at[pl.ds(base, BLK)], sidx[p])
                pltpu.sync_copy(cidx_h.at[pl.ds(base, BLK)], cidx[p])
                pltpu.sync_copy(eidx_h.at[pl.ds(base, BLK)], eidx[p])

                gathers = []
                for j in range(BLK // CH):
                    gathers.append(pltpu.async_copy(
                        sku_tab.at[sidx[p].at[pl.ds(j * CH, CH)]],
                        sbuf[p].at[pl.ds(j * CH, CH)],
                        gsem[p]))

                @pl.loop(0, BLK // L)
                def _(g):
                    rbase = g * L
                    rows = rows0 + rbase
                    ci = cidx[p][pl.ds(rbase, L)] * CAT_D
                    ei = eidx[p][pl.ds(rbase, L)] * EVT_D
                    for c in range(CAT_D):
                        vals = plsc.load_gather(cat_v, [ci + c])
                        plsc.store_scatter(
                            obuf[p],
                            [rows, jnp.full((L,), c, jnp.int32)],
                            vals)
                    for c in range(EVT_D):
                        vals = plsc.load_gather(evt_v, [ei + c])
                        plsc.store_scatter(
                            obuf[p],
                            [rows, jnp.full((L,), CAT_D + c, jnp.int32)],
                            vals)

                for g in gathers:
                    g.wait()
                pltpu.async_copy(
                    sbuf[p], out.at[pl.ds(base, BLK), pl.ds(0, SKU_D)], osem[p])
                pltpu.async_copy(
                    obuf[p], out.at[pl.ds(base, BLK), pl.ds(SKU_D, CAT_D + EVT_D)],
                    osem[p])

        for p in range(2):
            pltpu.make_async_copy(
                sbuf[p], out.at[pl.ds(0, BLK), pl.ds(0, SKU_D)], osem[p]).wait()
            pltpu.make_async_copy(
                obuf[p], out.at[pl.ds(0, BLK), pl.ds(SKU_D, CAT_D + EVT_D)],
                osem[p]).wait()

    return k


def kernel(sku, category, event_type, sku_table, category_table, event_type_table):
    B, H = sku.shape
    R = B * H
    sku_i = sku.reshape(R).astype(jnp.int32)
    cat_i = category.reshape(R).astype(jnp.int32)
    evt_i = event_type.reshape(R).astype(jnp.int32)
    out = _make_kernel(R)(
        sku_i, cat_i, evt_i,
        sku_table.astype(jnp.float32),
        category_table.reshape(CAT_V * CAT_D).astype(jnp.float32),
        event_type_table.reshape(EVT_V * EVT_D).astype(jnp.float32),
    )
    return out.reshape(B, H, OUT_D)
